# Initial kernel scaffold; baseline (speedup 1.0000x reference)
import jax
import jax.numpy as jnp
from jax.experimental import pallas as pl


def kernel(pred, target):
    raise NotImplementedError("write your pallas kernel here")


# speedup vs baseline: 1.2859x; 1.2859x over previous
import functools

import jax
import jax.numpy as jnp
from jax import lax
from jax.experimental import pallas as pl
from jax.experimental.pallas import tpu as pltpu
from jax.experimental.pallas import tpu_sc as plsc

B = 1024
C = 100000
SMOOTH = 0.1
CONF = 1.0 - SMOOTH
EPS = SMOOTH / (C - 1)

R = 32
BB = B // R


def _stats_body(x_ref, lse_ref, t_ref):
    x = x_ref[...]
    m = jnp.max(x, axis=1, keepdims=True)
    s = jnp.sum(jnp.exp(x - m), axis=1, keepdims=True)
    t = jnp.sum(x, axis=1, keepdims=True)
    lse_ref[0, :, :] = m + jnp.log(s)
    t_ref[0, :, :] = t


_stats = pl.pallas_call(
    _stats_body,
    grid=(BB,),
    in_specs=[pl.BlockSpec((R, C), lambda b: (b, 0))],
    out_specs=[
        pl.BlockSpec((1, R, 1), lambda b: (b, 0, 0)),
        pl.BlockSpec((1, R, 1), lambda b: (b, 0, 0)),
    ],
    out_shape=[
        jax.ShapeDtypeStruct((BB, R, 1), jnp.float32),
        jax.ShapeDtypeStruct((BB, R, 1), jnp.float32),
    ],
    compiler_params=pltpu.CompilerParams(
        dimension_semantics=("arbitrary",)),
)


NC = 2
NS = 16
NW = NC * NS
BPW = B // NW
L = 16


def _gather_body(pred_hbm, tgt_hbm, out_hbm, tgt_v, idx_v, val_v, sem):
    wid = lax.axis_index("s") * NC + lax.axis_index("c")
    base = wid * BPW
    pltpu.sync_copy(tgt_hbm.at[pl.ds(base, BPW)], tgt_v)
    for j in range(BPW // L):
        t16 = tgt_v[pl.ds(j * L, L)]
        rows = lax.iota(jnp.int32, L) + (base + j * L)
        idx_v[pl.ds(j * L, L)] = t16 + rows * C
    pltpu.async_copy(pred_hbm.at[idx_v], val_v, sem).wait()
    pltpu.sync_copy(val_v, out_hbm.at[pl.ds(base, BPW)])


@functools.cache
def _make_gather():
    return functools.partial(
        pl.kernel,
        mesh=plsc.VectorSubcoreMesh(core_axis_name="c", subcore_axis_name="s"),
        out_type=jax.ShapeDtypeStruct((B,), jnp.float32),
        scratch_types=[
            pltpu.VMEM((BPW,), jnp.int32),
            pltpu.VMEM((BPW,), jnp.int32),
            pltpu.VMEM((BPW,), jnp.float32),
            pltpu.SemaphoreType.DMA,
        ],
    )(_gather_body)


def kernel(pred, target):
    picked = _make_gather()(pred.reshape(-1), target)
    lse, t = _stats(pred)
    lse = lse.reshape(B)
    t = t.reshape(B)
    row = EPS * (t - C * lse) + (CONF - EPS) * (picked - lse)
    return -jnp.mean(row)


# speedup vs baseline: 2.4486x; 1.9042x over previous
import functools

import jax
import jax.numpy as jnp
from jax import lax
from jax.experimental import pallas as pl
from jax.experimental.pallas import tpu as pltpu
from jax.experimental.pallas import tpu_sc as plsc

B = 1024
C = 100000
SMOOTH = 0.1
CONF = 1.0 - SMOOTH
EPS = SMOOTH / (C - 1)

R = 32
BB = B // R


def _stats_body(x_ref, tgt_ref, lse_ref, t_ref, p_ref):
    x = x_ref[...]
    tgt = tgt_ref[0]
    m = jnp.max(x, axis=1, keepdims=True)
    s = jnp.sum(jnp.exp(x - m), axis=1, keepdims=True)
    t = jnp.sum(x, axis=1, keepdims=True)
    ids = lax.broadcasted_iota(jnp.int32, (R, C), 1)
    p = jnp.sum(jnp.where(ids == tgt, x, 0.0), axis=1, keepdims=True)
    lse_ref[0, :, :] = m + jnp.log(s)
    t_ref[0, :, :] = t
    p_ref[0, :, :] = p


_stats = pl.pallas_call(
    _stats_body,
    grid=(BB,),
    in_specs=[
        pl.BlockSpec((R, C), lambda b: (b, 0)),
        pl.BlockSpec((1, R, 1), lambda b: (b, 0, 0)),
    ],
    out_specs=[
        pl.BlockSpec((1, R, 1), lambda b: (b, 0, 0)),
        pl.BlockSpec((1, R, 1), lambda b: (b, 0, 0)),
        pl.BlockSpec((1, R, 1), lambda b: (b, 0, 0)),
    ],
    out_shape=[
        jax.ShapeDtypeStruct((BB, R, 1), jnp.float32),
        jax.ShapeDtypeStruct((BB, R, 1), jnp.float32),
        jax.ShapeDtypeStruct((BB, R, 1), jnp.float32),
    ],
    compiler_params=pltpu.CompilerParams(
        dimension_semantics=("parallel",)),
)


L = 16


def _combine_body(lse_hbm, t_hbm, p_hbm, out_hbm, lse_v, t_v, p_v, o_v, sem):
    cid = lax.axis_index("c")
    sid = lax.axis_index("s")

    @pl.when(jnp.logical_and(cid == 0, sid == 0))
    def _():
        pltpu.sync_copy(lse_hbm, lse_v)
        pltpu.sync_copy(t_hbm, t_v)
        pltpu.sync_copy(p_hbm, p_v)

        def body(j, acc):
            sl = pl.ds(j * L, L)
            lse = lse_v[sl]
            row = EPS * (t_v[sl] - C * lse) + (CONF - EPS) * (p_v[sl] - lse)
            return acc - row

        acc = lax.fori_loop(0, B // L, body, jnp.zeros((L,), jnp.float32))
        o_v[...] = acc * (1.0 / B)
        pltpu.sync_copy(o_v, out_hbm)


@functools.cache
def _make_combine():
    return functools.partial(
        pl.kernel,
        mesh=plsc.VectorSubcoreMesh(core_axis_name="c", subcore_axis_name="s"),
        out_type=jax.ShapeDtypeStruct((L,), jnp.float32),
        scratch_types=[
            pltpu.VMEM((B,), jnp.float32),
            pltpu.VMEM((B,), jnp.float32),
            pltpu.VMEM((B,), jnp.float32),
            pltpu.VMEM((L,), jnp.float32),
            pltpu.SemaphoreType.DMA,
        ],
    )(_combine_body)


def kernel(pred, target):
    tgt = target.reshape(BB, R, 1)
    lse, t, p = _stats(pred, tgt)
    out = _make_combine()(lse.reshape(B), t.reshape(B), p.reshape(B))
    return jnp.sum(out)
